# baseline (speedup 1.0000x reference)
import jax
import jax.numpy as jnp
from jax import lax
from jax.experimental import pallas as pl
from jax.experimental.pallas import tpu as pltpu
from jax.experimental.pallas import tpu_sc as plsc

_B = 16
_N = 8400
_C = 84
_CHUNK = 4208
_BASE1 = _N - _CHUNK
_NV = _CHUNK // 16
_INV_W = 1.0 / 80.0
_INV_H = 1.0 / 80.0


def _roi_body(yolo_hbm, out_hbm, cx_v, cy_v, w_v, h_v, out_v, sem):
    cid = lax.axis_index("c")
    sid = lax.axis_index("s")
    wid = sid * 2 + cid
    b = wid // 2
    half = wid - 2 * b
    base = half * _BASE1
    row0 = b * _C * _N + base
    d0 = pltpu.async_copy(yolo_hbm.at[pl.ds(row0, _CHUNK)], cx_v, sem)
    d1 = pltpu.async_copy(yolo_hbm.at[pl.ds(row0 + _N, _CHUNK)], cy_v, sem)
    d2 = pltpu.async_copy(yolo_hbm.at[pl.ds(row0 + 2 * _N, _CHUNK)], w_v, sem)
    d3 = pltpu.async_copy(yolo_hbm.at[pl.ds(row0 + 3 * _N, _CHUNK)], h_v, sem)
    d0.wait()
    d1.wait()
    d2.wait()
    d3.wait()

    iota5 = jnp.arange(16, dtype=jnp.int32) * 5
    bvec = jnp.broadcast_to(b.astype(jnp.float32), (16,))

    def step(i, carry):
        off = i * 16
        cx = cx_v[pl.ds(off, 16)]
        cy = cy_v[pl.ds(off, 16)]
        w = w_v[pl.ds(off, 16)]
        h = h_v[pl.ds(off, 16)]
        hw = w * 0.5
        hh = h * 0.5
        x1 = (cx - hw) * _INV_W
        x2 = (cx + hw) * _INV_W
        y1 = (cy - hh) * _INV_H
        y2 = (cy + hh) * _INV_H
        xmin = jnp.clip(jnp.minimum(x1, x2), 0.0, 1.0)
        ymin = jnp.clip(jnp.minimum(y1, y2), 0.0, 1.0)
        xmax = jnp.clip(jnp.maximum(x1, x2), 0.0, 1.0)
        ymax = jnp.clip(jnp.maximum(y1, y2), 0.0, 1.0)
        idx0 = iota5 + off * 5
        plsc.store_scatter(out_v, [idx0], bvec)
        plsc.store_scatter(out_v, [idx0 + 1], xmin)
        plsc.store_scatter(out_v, [idx0 + 2], ymin)
        plsc.store_scatter(out_v, [idx0 + 3], xmax)
        plsc.store_scatter(out_v, [idx0 + 4], ymax)
        return carry

    lax.fori_loop(0, _NV, step, 0)
    pltpu.sync_copy(out_v, out_hbm.at[pl.ds((b * _N + base) * 5, _CHUNK * 5)])


def kernel(yolo_output, input_images_or_features):
    del input_images_or_features
    yolo_flat = yolo_output.reshape(-1)
    mesh = plsc.VectorSubcoreMesh(core_axis_name="c", subcore_axis_name="s")
    run = pl.kernel(
        _roi_body,
        out_type=jax.ShapeDtypeStruct((_B * _N * 5,), jnp.float32),
        mesh=mesh,
        scratch_types=[
            pltpu.VMEM((_CHUNK,), jnp.float32),
            pltpu.VMEM((_CHUNK,), jnp.float32),
            pltpu.VMEM((_CHUNK,), jnp.float32),
            pltpu.VMEM((_CHUNK,), jnp.float32),
            pltpu.VMEM((_CHUNK * 5,), jnp.float32),
            pltpu.SemaphoreType.DMA,
        ],
        compiler_params=pltpu.CompilerParams(needs_layout_passes=False),
    )
    out = run(yolo_flat)
    return out.reshape(_B * _N, 5)
